# Initial kernel scaffold; baseline (speedup 1.0000x reference)
import functools

import jax
import jax.numpy as jnp
from jax import lax
from jax.experimental import pallas as pl
from jax.experimental.pallas import tpu as pltpu
from jax.experimental.pallas import tpu_sc as plsc

N_NODES = 10000
DIM = 128
N_PAD = 10240
NC, NS = 2, 16
NW = NC * NS
CHUNK = 128
CPW = 79
E_PAD = NW * CPW * CHUNK
RPT = N_PAD // NS
RB = 1280


def _deg_body(dst2d, zeros1, degp, didx_v, ones_v, acc_s, sem):
    cid = lax.axis_index("c")
    sid = lax.axis_index("s")
    wid = sid * NC + cid
    for k in range(CHUNK // 16):
        ones_v[pl.ds(16 * k, 16)] = jnp.full((16,), 1.0, jnp.float32)
    pltpu.sync_copy(zeros1.at[pl.ds(sid * RPT, RPT)],
                    acc_s.at[pl.ds(sid * RPT, RPT)])
    plsc.subcore_barrier()
    pltpu.sync_copy(dst2d.at[pl.ds(wid * CPW, CPW)], didx_v)

    def body(j, carry):
        pltpu.sync_copy(ones_v, acc_s.at[didx_v.at[j]], add=True)
        return carry

    lax.fori_loop(0, CPW, body, 0)
    plsc.subcore_barrier()
    pltpu.sync_copy(acc_s.at[pl.ds(sid * RPT, RPT)],
                    degp.at[cid, pl.ds(sid * RPT, RPT)])


def _deg(dst2d, zeros1):
    mesh = plsc.VectorSubcoreMesh(core_axis_name="c", subcore_axis_name="s")
    return pl.kernel(
        _deg_body,
        out_type=jax.ShapeDtypeStruct((NC, N_PAD), jnp.float32),
        mesh=mesh,
        scratch_types=[
            pltpu.VMEM((CPW, CHUNK), jnp.int32),
            pltpu.VMEM((CHUNK,), jnp.float32),
            pltpu.VMEM_SHARED((N_PAD,), jnp.float32),
            pltpu.SemaphoreType.DMA,
        ],
    )(dst2d, zeros1)


def _agg_body(g, src2d, dst2d, zeros2, accp,
              sidx_v, didx_v, rows0, rows1, acc_s, sem0, sem1):
    cid = lax.axis_index("c")
    sid = lax.axis_index("s")
    wid = sid * NC + cid
    base = wid * CPW
    pltpu.sync_copy(zeros2.at[pl.ds(sid * RPT, RPT)],
                    acc_s.at[pl.ds(sid * RPT, RPT)])
    pltpu.sync_copy(src2d.at[pl.ds(base, CPW)], sidx_v)
    pltpu.sync_copy(dst2d.at[pl.ds(base, CPW)], didx_v)
    plsc.subcore_barrier()

    pltpu.async_copy(g.at[sidx_v.at[0]], rows0, sem0)

    def body(j2, carry):
        c0 = 2 * j2
        pltpu.async_copy(g.at[sidx_v.at[c0 + 1]], rows1, sem1)
        pltpu.make_async_copy(g.at[sidx_v.at[c0]], rows0, sem0).wait()
        pltpu.sync_copy(rows0, acc_s.at[didx_v.at[c0]], add=True)
        pltpu.async_copy(g.at[sidx_v.at[c0 + 2]], rows0, sem0)
        pltpu.make_async_copy(g.at[sidx_v.at[c0 + 1]], rows1, sem1).wait()
        pltpu.sync_copy(rows1, acc_s.at[didx_v.at[c0 + 1]], add=True)
        return carry

    lax.fori_loop(0, (CPW - 1) // 2, body, 0)
    pltpu.make_async_copy(g.at[sidx_v.at[CPW - 1]], rows0, sem0).wait()
    pltpu.sync_copy(rows0, acc_s.at[didx_v.at[CPW - 1]], add=True)

    plsc.subcore_barrier()
    pltpu.sync_copy(acc_s.at[pl.ds(sid * RPT, RPT)],
                    accp.at[cid, pl.ds(sid * RPT, RPT)])


def _agg(g, src2d, dst2d, zeros2):
    mesh = plsc.VectorSubcoreMesh(core_axis_name="c", subcore_axis_name="s")
    return pl.kernel(
        _agg_body,
        out_type=jax.ShapeDtypeStruct((NC, N_PAD, DIM), jnp.float32),
        mesh=mesh,
        scratch_types=[
            pltpu.VMEM((CPW, CHUNK), jnp.int32),
            pltpu.VMEM((CPW, CHUNK), jnp.int32),
            pltpu.VMEM((CHUNK, DIM), jnp.float32),
            pltpu.VMEM((CHUNK, DIM), jnp.float32),
            pltpu.VMEM_SHARED((N_PAD, DIM), jnp.float32),
            pltpu.SemaphoreType.DMA,
            pltpu.SemaphoreType.DMA,
        ],
    )(g, src2d, dst2d, zeros2)


def _lin_body(x_ref, w_ref, degp_ref, g_ref):
    deg = degp_ref[0, :] + degp_ref[1, :] + 1.0
    dis = lax.rsqrt(deg)
    h = lax.dot_general(x_ref[...], w_ref[...], (((1,), (1,)), ((), ())),
                        preferred_element_type=jnp.float32,
                        precision=lax.Precision.HIGHEST)
    g_ref[...] = h * dis[:, None]


def _lin(x_pad, W, degp):
    return pl.pallas_call(
        _lin_body,
        grid=(N_PAD // RB,),
        in_specs=[
            pl.BlockSpec((RB, DIM), lambda i: (i, 0)),
            pl.BlockSpec((DIM, DIM), lambda i: (0, 0)),
            pl.BlockSpec((NC, RB), lambda i: (0, i)),
        ],
        out_specs=pl.BlockSpec((RB, DIM), lambda i: (i, 0)),
        out_shape=jax.ShapeDtypeStruct((N_PAD, DIM), jnp.float32),
    )(x_pad, W, degp)


def _out_body(accp_ref, g_ref, degp_ref, b_ref, o_ref):
    deg = degp_ref[0, :] + degp_ref[1, :] + 1.0
    dis = lax.rsqrt(deg)
    s = accp_ref[0] + accp_ref[1] + g_ref[...]
    o_ref[...] = s * dis[:, None] + b_ref[...]


def _out(accp, g, degp, b2):
    return pl.pallas_call(
        _out_body,
        grid=(N_PAD // RB,),
        in_specs=[
            pl.BlockSpec((NC, RB, DIM), lambda i: (0, i, 0)),
            pl.BlockSpec((RB, DIM), lambda i: (i, 0)),
            pl.BlockSpec((NC, RB), lambda i: (0, i)),
            pl.BlockSpec((1, DIM), lambda i: (0, 0)),
        ],
        out_specs=pl.BlockSpec((RB, DIM), lambda i: (i, 0)),
        out_shape=jax.ShapeDtypeStruct((N_PAD, DIM), jnp.float32),
    )(accp, g, degp, b2)


def kernel(x, edge_index, W, b):
    n_edges = edge_index.shape[1]
    src = edge_index[0].astype(jnp.int32)
    dst = edge_index[1].astype(jnp.int32)
    n_extra = E_PAD - n_edges
    pad_idx = (jnp.arange(n_extra, dtype=jnp.int32) % (N_PAD - N_NODES)) + N_NODES
    src2d = jnp.concatenate([src, pad_idx]).reshape(NW * CPW, CHUNK)
    dst2d = jnp.concatenate([dst, pad_idx]).reshape(NW * CPW, CHUNK)
    x_pad = jnp.concatenate([x, jnp.zeros((N_PAD - N_NODES, DIM), x.dtype)])
    zeros1 = jnp.zeros((N_PAD,), jnp.float32)
    zeros2 = jnp.zeros((N_PAD, DIM), jnp.float32)

    degp = _deg(dst2d, zeros1)
    g = _lin(x_pad, W, degp)
    accp = _agg(g, src2d, dst2d, zeros2)
    out_pad = _out(accp, g, degp, b.reshape(1, DIM))
    return out_pad[:N_NODES]


# speedup vs baseline: 42.0748x; 42.0748x over previous
import functools

import jax
import jax.numpy as jnp
from jax import lax
from jax.experimental import pallas as pl
from jax.experimental.pallas import tpu as pltpu
from jax.experimental.pallas import tpu_sc as plsc

N_NODES = 10000
DIM = 128
N_PAD = 10240
NC, NS = 2, 16
NW = NC * NS
CHUNK = 128
CPW = 80
E_PAD = NW * CPW * CHUNK
RPT = N_PAD // NS
HC = CPW // 2
RB = 1280


def _deg_body(dst2d, zeros1, degp, didx_v, ones_v, acc_s, sem):
    cid = lax.axis_index("c")
    sid = lax.axis_index("s")
    wid = sid * NC + cid
    for k in range(CHUNK // 16):
        ones_v[pl.ds(16 * k, 16)] = jnp.full((16,), 1.0, jnp.float32)
    pltpu.sync_copy(zeros1.at[pl.ds(sid * RPT, RPT)],
                    acc_s.at[pl.ds(sid * RPT, RPT)])
    plsc.subcore_barrier()
    base = pl.multiple_of(wid * CPW, 8)
    pltpu.sync_copy(dst2d.at[pl.ds(base, CPW)], didx_v)

    def body(j, carry):
        pltpu.sync_copy(ones_v, acc_s.at[didx_v.at[j]], add=True)
        return carry

    lax.fori_loop(0, CPW, body, 0)
    plsc.subcore_barrier()
    pltpu.sync_copy(acc_s.at[pl.ds(sid * RPT, RPT)],
                    degp.at[cid, pl.ds(sid * RPT, RPT)])


def _deg(dst2d, zeros1):
    mesh = plsc.VectorSubcoreMesh(core_axis_name="c", subcore_axis_name="s")
    return pl.kernel(
        _deg_body,
        out_type=jax.ShapeDtypeStruct((NC, N_PAD), jnp.float32),
        mesh=mesh,
        scratch_types=[
            pltpu.VMEM((CPW, CHUNK), jnp.int32),
            pltpu.VMEM((CHUNK,), jnp.float32),
            pltpu.VMEM_SHARED((N_PAD,), jnp.float32),
            pltpu.SemaphoreType.DMA,
        ],
    )(dst2d, zeros1)


def _agg_body(g, src2d, dst2d, zeros2, accp,
              sidx_v, didx_v, rows0, rows1, acc_s, sem0, sem1):
    cid = lax.axis_index("c")
    sid = lax.axis_index("s")
    wid = sid * NC + cid
    pltpu.sync_copy(zeros2.at[pl.ds(sid * RPT, RPT)],
                    acc_s.at[pl.ds(sid * RPT, RPT)])
    plsc.subcore_barrier()

    for h in range(CPW // HC):
        base = pl.multiple_of(wid * CPW + h * HC, 8)
        pltpu.sync_copy(src2d.at[pl.ds(base, HC)], sidx_v)
        pltpu.sync_copy(dst2d.at[pl.ds(base, HC)], didx_v)

        pltpu.async_copy(g.at[sidx_v.at[0]], rows0, sem0)

        def body(j2, carry):
            c0 = 2 * j2
            pltpu.async_copy(g.at[sidx_v.at[c0 + 1]], rows1, sem1)
            pltpu.make_async_copy(g.at[sidx_v.at[c0]], rows0, sem0).wait()
            pltpu.sync_copy(rows0, acc_s.at[didx_v.at[c0]], add=True)
            pltpu.async_copy(g.at[sidx_v.at[c0 + 2]], rows0, sem0)
            pltpu.make_async_copy(g.at[sidx_v.at[c0 + 1]], rows1, sem1).wait()
            pltpu.sync_copy(rows1, acc_s.at[didx_v.at[c0 + 1]], add=True)
            return carry

        lax.fori_loop(0, HC // 2 - 1, body, 0)
        pltpu.async_copy(g.at[sidx_v.at[HC - 1]], rows1, sem1)
        pltpu.make_async_copy(g.at[sidx_v.at[HC - 2]], rows0, sem0).wait()
        pltpu.sync_copy(rows0, acc_s.at[didx_v.at[HC - 2]], add=True)
        pltpu.make_async_copy(g.at[sidx_v.at[HC - 1]], rows1, sem1).wait()
        pltpu.sync_copy(rows1, acc_s.at[didx_v.at[HC - 1]], add=True)

    plsc.subcore_barrier()
    pltpu.sync_copy(acc_s.at[pl.ds(sid * RPT, RPT)],
                    accp.at[cid, pl.ds(sid * RPT, RPT)])


def _agg(g, src2d, dst2d, zeros2):
    mesh = plsc.VectorSubcoreMesh(core_axis_name="c", subcore_axis_name="s")
    return pl.kernel(
        _agg_body,
        out_type=jax.ShapeDtypeStruct((NC, N_PAD, DIM), jnp.float32),
        mesh=mesh,
        scratch_types=[
            pltpu.VMEM((HC, CHUNK), jnp.int32),
            pltpu.VMEM((HC, CHUNK), jnp.int32),
            pltpu.VMEM((CHUNK, DIM), jnp.float32),
            pltpu.VMEM((CHUNK, DIM), jnp.float32),
            pltpu.VMEM_SHARED((N_PAD, DIM), jnp.float32),
            pltpu.SemaphoreType.DMA,
            pltpu.SemaphoreType.DMA,
        ],
    )(g, src2d, dst2d, zeros2)


def _lin_body(x_ref, w_ref, degp_ref, g_ref):
    deg = degp_ref[0, :] + degp_ref[1, :] + 1.0
    dis = lax.rsqrt(deg)
    h = lax.dot_general(x_ref[...], w_ref[...], (((1,), (1,)), ((), ())),
                        preferred_element_type=jnp.float32,
                        precision=lax.Precision.HIGHEST)
    g_ref[...] = h * dis[:, None]


def _lin(x_pad, W, degp):
    return pl.pallas_call(
        _lin_body,
        grid=(N_PAD // RB,),
        in_specs=[
            pl.BlockSpec((RB, DIM), lambda i: (i, 0)),
            pl.BlockSpec((DIM, DIM), lambda i: (0, 0)),
            pl.BlockSpec((NC, RB), lambda i: (0, i)),
        ],
        out_specs=pl.BlockSpec((RB, DIM), lambda i: (i, 0)),
        out_shape=jax.ShapeDtypeStruct((N_PAD, DIM), jnp.float32),
    )(x_pad, W, degp)


def _out_body(accp_ref, g_ref, degp_ref, b_ref, o_ref):
    deg = degp_ref[0, :] + degp_ref[1, :] + 1.0
    dis = lax.rsqrt(deg)
    s = accp_ref[0] + accp_ref[1] + g_ref[...]
    o_ref[...] = s * dis[:, None] + b_ref[...]


def _out(accp, g, degp, b2):
    return pl.pallas_call(
        _out_body,
        grid=(N_PAD // RB,),
        in_specs=[
            pl.BlockSpec((NC, RB, DIM), lambda i: (0, i, 0)),
            pl.BlockSpec((RB, DIM), lambda i: (i, 0)),
            pl.BlockSpec((NC, RB), lambda i: (0, i)),
            pl.BlockSpec((1, DIM), lambda i: (0, 0)),
        ],
        out_specs=pl.BlockSpec((RB, DIM), lambda i: (i, 0)),
        out_shape=jax.ShapeDtypeStruct((N_PAD, DIM), jnp.float32),
    )(accp, g, degp, b2)


def kernel(x, edge_index, W, b):
    n_edges = edge_index.shape[1]
    src = edge_index[0].astype(jnp.int32)
    dst = edge_index[1].astype(jnp.int32)
    n_extra = E_PAD - n_edges
    pad_idx = (jnp.arange(n_extra, dtype=jnp.int32) % (N_PAD - N_NODES)) + N_NODES
    src2d = jnp.concatenate([src, pad_idx]).reshape(NW * CPW, CHUNK)
    dst2d = jnp.concatenate([dst, pad_idx]).reshape(NW * CPW, CHUNK)
    x_pad = jnp.concatenate([x, jnp.zeros((N_PAD - N_NODES, DIM), x.dtype)])
    zeros1 = jnp.zeros((N_PAD,), jnp.float32)
    zeros2 = jnp.zeros((N_PAD, DIM), jnp.float32)

    degp = _deg(dst2d, zeros1)
    g = _lin(x_pad, W, degp)
    accp = _agg(g, src2d, dst2d, zeros2)
    out_pad = _out(accp, g, degp, b.reshape(1, DIM))
    return out_pad[:N_NODES]


# speedup vs baseline: 42.2970x; 1.0053x over previous
import functools

import jax
import jax.numpy as jnp
from jax import lax
from jax.experimental import pallas as pl
from jax.experimental.pallas import tpu as pltpu
from jax.experimental.pallas import tpu_sc as plsc

N_NODES = 10000
DIM = 128
N_PAD = 10240
NC, NS = 2, 16
NW = NC * NS
CHUNK = 128
CPW = 80
E_PAD = NW * CPW * CHUNK
RPT = N_PAD // NS
HC = CPW // 2
RB = 1280


def _deg_body(dst2d, zrow, degp, didx_v, ones_v, acc_s, sem):
    cid = lax.axis_index("c")
    sid = lax.axis_index("s")
    wid = sid * NC + cid
    for k in range(CHUNK // 16):
        ones_v[pl.ds(16 * k, 16)] = jnp.full((16,), 1.0, jnp.float32)
    pltpu.sync_copy(zrow, acc_s.at[pl.ds(sid * RPT, RPT)])
    plsc.subcore_barrier()
    base = pl.multiple_of(wid * CPW, 8)
    pltpu.sync_copy(dst2d.at[pl.ds(base, CPW)], didx_v)

    def body(j, carry):
        pltpu.sync_copy(ones_v, acc_s.at[didx_v.at[j]], add=True)
        return carry

    lax.fori_loop(0, CPW, body, 0)
    plsc.subcore_barrier()
    pltpu.sync_copy(acc_s.at[pl.ds(sid * RPT, RPT)],
                    degp.at[cid, pl.ds(sid * RPT, RPT)])


def _deg(dst2d, zrow):
    mesh = plsc.VectorSubcoreMesh(core_axis_name="c", subcore_axis_name="s")
    return pl.kernel(
        _deg_body,
        out_type=jax.ShapeDtypeStruct((NC, N_PAD), jnp.float32),
        mesh=mesh,
        scratch_types=[
            pltpu.VMEM((CPW, CHUNK), jnp.int32),
            pltpu.VMEM((CHUNK,), jnp.float32),
            pltpu.VMEM_SHARED((N_PAD,), jnp.float32),
            pltpu.SemaphoreType.DMA,
        ],
    )(dst2d, zrow)


def _agg_body(g, src2d, dst2d, zblk, accp,
              sidx_v, didx_v, rows0, rows1, acc_s, sem0, sem1):
    cid = lax.axis_index("c")
    sid = lax.axis_index("s")
    wid = sid * NC + cid
    rbase = pl.multiple_of(sid * RPT, 8)
    pltpu.sync_copy(zblk, rows0)
    for k in range(RPT // CHUNK):
        pltpu.sync_copy(rows0, acc_s.at[pl.ds(rbase + k * CHUNK, CHUNK)])
    plsc.subcore_barrier()

    for h in range(CPW // HC):
        base = pl.multiple_of(wid * CPW + h * HC, 8)
        pltpu.sync_copy(src2d.at[pl.ds(base, HC)], sidx_v)
        pltpu.sync_copy(dst2d.at[pl.ds(base, HC)], didx_v)

        pltpu.async_copy(g.at[sidx_v.at[0]], rows0, sem0)

        def body(j2, carry):
            c0 = 2 * j2
            pltpu.async_copy(g.at[sidx_v.at[c0 + 1]], rows1, sem1)
            pltpu.make_async_copy(g.at[sidx_v.at[c0]], rows0, sem0).wait()
            pltpu.sync_copy(rows0, acc_s.at[didx_v.at[c0]], add=True)
            pltpu.async_copy(g.at[sidx_v.at[c0 + 2]], rows0, sem0)
            pltpu.make_async_copy(g.at[sidx_v.at[c0 + 1]], rows1, sem1).wait()
            pltpu.sync_copy(rows1, acc_s.at[didx_v.at[c0 + 1]], add=True)
            return carry

        lax.fori_loop(0, HC // 2 - 1, body, 0)
        pltpu.async_copy(g.at[sidx_v.at[HC - 1]], rows1, sem1)
        pltpu.make_async_copy(g.at[sidx_v.at[HC - 2]], rows0, sem0).wait()
        pltpu.sync_copy(rows0, acc_s.at[didx_v.at[HC - 2]], add=True)
        pltpu.make_async_copy(g.at[sidx_v.at[HC - 1]], rows1, sem1).wait()
        pltpu.sync_copy(rows1, acc_s.at[didx_v.at[HC - 1]], add=True)

    plsc.subcore_barrier()
    pltpu.sync_copy(acc_s.at[pl.ds(sid * RPT, RPT)],
                    accp.at[cid, pl.ds(sid * RPT, RPT)])


def _agg(g, src2d, dst2d, zblk):
    mesh = plsc.VectorSubcoreMesh(core_axis_name="c", subcore_axis_name="s")
    return pl.kernel(
        _agg_body,
        out_type=jax.ShapeDtypeStruct((NC, N_PAD, DIM), jnp.float32),
        mesh=mesh,
        scratch_types=[
            pltpu.VMEM((HC, CHUNK), jnp.int32),
            pltpu.VMEM((HC, CHUNK), jnp.int32),
            pltpu.VMEM((CHUNK, DIM), jnp.float32),
            pltpu.VMEM((CHUNK, DIM), jnp.float32),
            pltpu.VMEM_SHARED((N_PAD, DIM), jnp.float32),
            pltpu.SemaphoreType.DMA,
            pltpu.SemaphoreType.DMA,
        ],
    )(g, src2d, dst2d, zblk)


def _lin_body(x_ref, w_ref, degp_ref, g_ref):
    deg = degp_ref[0, :] + degp_ref[1, :] + 1.0
    dis = lax.rsqrt(deg)
    h = lax.dot_general(x_ref[...], w_ref[...], (((1,), (1,)), ((), ())),
                        preferred_element_type=jnp.float32,
                        precision=lax.Precision.HIGHEST)
    g_ref[...] = h * dis[:, None]


def _lin(x_pad, W, degp):
    return pl.pallas_call(
        _lin_body,
        grid=(N_PAD // RB,),
        in_specs=[
            pl.BlockSpec((RB, DIM), lambda i: (i, 0)),
            pl.BlockSpec((DIM, DIM), lambda i: (0, 0)),
            pl.BlockSpec((NC, RB), lambda i: (0, i)),
        ],
        out_specs=pl.BlockSpec((RB, DIM), lambda i: (i, 0)),
        out_shape=jax.ShapeDtypeStruct((N_PAD, DIM), jnp.float32),
    )(x_pad, W, degp)


def _out_body(accp_ref, g_ref, degp_ref, b_ref, o_ref):
    deg = degp_ref[0, :] + degp_ref[1, :] + 1.0
    dis = lax.rsqrt(deg)
    s = accp_ref[0] + accp_ref[1] + g_ref[...]
    o_ref[...] = s * dis[:, None] + b_ref[...]


def _out(accp, g, degp, b2):
    return pl.pallas_call(
        _out_body,
        grid=(N_PAD // RB,),
        in_specs=[
            pl.BlockSpec((NC, RB, DIM), lambda i: (0, i, 0)),
            pl.BlockSpec((RB, DIM), lambda i: (i, 0)),
            pl.BlockSpec((NC, RB), lambda i: (0, i)),
            pl.BlockSpec((1, DIM), lambda i: (0, 0)),
        ],
        out_specs=pl.BlockSpec((RB, DIM), lambda i: (i, 0)),
        out_shape=jax.ShapeDtypeStruct((N_PAD, DIM), jnp.float32),
    )(accp, g, degp, b2)


def kernel(x, edge_index, W, b):
    n_edges = edge_index.shape[1]
    src = edge_index[0].astype(jnp.int32)
    dst = edge_index[1].astype(jnp.int32)
    n_extra = E_PAD - n_edges
    pad_idx = (jnp.arange(n_extra, dtype=jnp.int32) % (N_PAD - N_NODES)) + N_NODES
    src2d = jnp.concatenate([src, pad_idx]).reshape(NW * CPW, CHUNK)
    dst2d = jnp.concatenate([dst, pad_idx]).reshape(NW * CPW, CHUNK)
    x_pad = jnp.concatenate([x, jnp.zeros((N_PAD - N_NODES, DIM), x.dtype)])
    zrow = jnp.zeros((RPT,), jnp.float32)
    zblk = jnp.zeros((CHUNK, DIM), jnp.float32)

    degp = _deg(dst2d, zrow)
    g = _lin(x_pad, W, degp)
    accp = _agg(g, src2d, dst2d, zblk)
    out_pad = _out(accp, g, degp, b.reshape(1, DIM))
    return out_pad[:N_NODES]


# speedup vs baseline: 43.4258x; 1.0267x over previous
import functools

import jax
import jax.numpy as jnp
from jax import lax
from jax.experimental import pallas as pl
from jax.experimental.pallas import tpu as pltpu
from jax.experimental.pallas import tpu_sc as plsc

N_NODES = 10000
DIM = 128
N_PAD = 10240
NC, NS = 2, 16
NW = NC * NS
CHUNK = 128
CPW = 80
E_PAD = NW * CPW * CHUNK
RPT = N_PAD // NS
HC = CPW // 2
RB = 1280
DW = 16


def _deg_body(dst2d, zrow, degp, didx_v, ones_v, acc_s, sem):
    cid = lax.axis_index("c")
    sid = lax.axis_index("s")
    wid = sid * NC + cid
    for k in range(CHUNK // 16):
        ones_v[pl.ds(16 * k, 16)] = jnp.full((16,), 1.0, jnp.float32)
    pltpu.sync_copy(zrow, acc_s.at[pl.ds(sid * RPT, RPT)])
    plsc.subcore_barrier()
    base = pl.multiple_of(wid * CPW, 8)
    pltpu.sync_copy(dst2d.at[pl.ds(base, CPW)], didx_v)

    for j in range(DW):
        pltpu.async_copy(ones_v, acc_s.at[didx_v.at[j]], sem, add=True)

    def body(j, carry):
        pltpu.make_async_copy(ones_v, acc_s.at[didx_v.at[0]], sem).wait()
        pltpu.async_copy(ones_v, acc_s.at[didx_v.at[j + DW]], sem, add=True)
        return carry

    lax.fori_loop(0, CPW - DW, body, 0)

    def drain(j, carry):
        pltpu.make_async_copy(ones_v, acc_s.at[didx_v.at[0]], sem).wait()
        return carry

    lax.fori_loop(0, DW, drain, 0)
    plsc.subcore_barrier()
    pltpu.sync_copy(acc_s.at[pl.ds(sid * RPT, RPT)],
                    degp.at[cid, pl.ds(sid * RPT, RPT)])


def _deg(dst2d, zrow):
    mesh = plsc.VectorSubcoreMesh(core_axis_name="c", subcore_axis_name="s")
    return pl.kernel(
        _deg_body,
        out_type=jax.ShapeDtypeStruct((NC, N_PAD), jnp.float32),
        mesh=mesh,
        scratch_types=[
            pltpu.VMEM((CPW, CHUNK), jnp.int32),
            pltpu.VMEM((CHUNK,), jnp.float32),
            pltpu.VMEM_SHARED((N_PAD,), jnp.float32),
            pltpu.SemaphoreType.DMA,
        ],
    )(dst2d, zrow)


def _agg_body(g, src2d, dst2d, zblk, accp,
              sidx_v, didx_v, rows0, rows1, acc_s, sem0, sem1):
    cid = lax.axis_index("c")
    sid = lax.axis_index("s")
    wid = sid * NC + cid
    rbase = pl.multiple_of(sid * RPT, 8)
    pltpu.sync_copy(zblk, rows0)
    for k in range(RPT // CHUNK):
        pltpu.sync_copy(rows0, acc_s.at[pl.ds(rbase + k * CHUNK, CHUNK)])
    plsc.subcore_barrier()

    for h in range(CPW // HC):
        base = pl.multiple_of(wid * CPW + h * HC, 8)
        pltpu.sync_copy(src2d.at[pl.ds(base, HC)], sidx_v)
        pltpu.sync_copy(dst2d.at[pl.ds(base, HC)], didx_v)

        pltpu.async_copy(g.at[sidx_v.at[0]], rows0, sem0)

        def body(j2, carry):
            c0 = 2 * j2
            pltpu.async_copy(g.at[sidx_v.at[c0 + 1]], rows1, sem1)
            pltpu.make_async_copy(g.at[sidx_v.at[c0]], rows0, sem0).wait()
            pltpu.sync_copy(rows0, acc_s.at[didx_v.at[c0]], add=True)
            pltpu.async_copy(g.at[sidx_v.at[c0 + 2]], rows0, sem0)
            pltpu.make_async_copy(g.at[sidx_v.at[c0 + 1]], rows1, sem1).wait()
            pltpu.sync_copy(rows1, acc_s.at[didx_v.at[c0 + 1]], add=True)
            return carry

        lax.fori_loop(0, HC // 2 - 1, body, 0)
        pltpu.async_copy(g.at[sidx_v.at[HC - 1]], rows1, sem1)
        pltpu.make_async_copy(g.at[sidx_v.at[HC - 2]], rows0, sem0).wait()
        pltpu.sync_copy(rows0, acc_s.at[didx_v.at[HC - 2]], add=True)
        pltpu.make_async_copy(g.at[sidx_v.at[HC - 1]], rows1, sem1).wait()
        pltpu.sync_copy(rows1, acc_s.at[didx_v.at[HC - 1]], add=True)

    plsc.subcore_barrier()
    pltpu.sync_copy(acc_s.at[pl.ds(sid * RPT, RPT)],
                    accp.at[cid, pl.ds(sid * RPT, RPT)])


def _agg(g, src2d, dst2d, zblk):
    mesh = plsc.VectorSubcoreMesh(core_axis_name="c", subcore_axis_name="s")
    return pl.kernel(
        _agg_body,
        out_type=jax.ShapeDtypeStruct((NC, N_PAD, DIM), jnp.float32),
        mesh=mesh,
        scratch_types=[
            pltpu.VMEM((HC, CHUNK), jnp.int32),
            pltpu.VMEM((HC, CHUNK), jnp.int32),
            pltpu.VMEM((CHUNK, DIM), jnp.float32),
            pltpu.VMEM((CHUNK, DIM), jnp.float32),
            pltpu.VMEM_SHARED((N_PAD, DIM), jnp.float32),
            pltpu.SemaphoreType.DMA,
            pltpu.SemaphoreType.DMA,
        ],
    )(g, src2d, dst2d, zblk)


def _lin_body(x_ref, w_ref, degp_ref, g_ref):
    deg = degp_ref[0, :] + degp_ref[1, :] + 1.0
    dis = lax.rsqrt(deg)
    h = lax.dot_general(x_ref[...], w_ref[...], (((1,), (1,)), ((), ())),
                        preferred_element_type=jnp.float32,
                        precision=lax.Precision.HIGHEST)
    g_ref[...] = h * dis[:, None]


def _lin(x_pad, W, degp):
    return pl.pallas_call(
        _lin_body,
        grid=(N_PAD // RB,),
        in_specs=[
            pl.BlockSpec((RB, DIM), lambda i: (i, 0)),
            pl.BlockSpec((DIM, DIM), lambda i: (0, 0)),
            pl.BlockSpec((NC, RB), lambda i: (0, i)),
        ],
        out_specs=pl.BlockSpec((RB, DIM), lambda i: (i, 0)),
        out_shape=jax.ShapeDtypeStruct((N_PAD, DIM), jnp.float32),
    )(x_pad, W, degp)


def _out_body(accp_ref, g_ref, degp_ref, b_ref, o_ref):
    deg = degp_ref[0, :] + degp_ref[1, :] + 1.0
    dis = lax.rsqrt(deg)
    s = accp_ref[0] + accp_ref[1] + g_ref[...]
    o_ref[...] = s * dis[:, None] + b_ref[...]


def _out(accp, g, degp, b2):
    return pl.pallas_call(
        _out_body,
        grid=(N_PAD // RB,),
        in_specs=[
            pl.BlockSpec((NC, RB, DIM), lambda i: (0, i, 0)),
            pl.BlockSpec((RB, DIM), lambda i: (i, 0)),
            pl.BlockSpec((NC, RB), lambda i: (0, i)),
            pl.BlockSpec((1, DIM), lambda i: (0, 0)),
        ],
        out_specs=pl.BlockSpec((RB, DIM), lambda i: (i, 0)),
        out_shape=jax.ShapeDtypeStruct((N_PAD, DIM), jnp.float32),
    )(accp, g, degp, b2)


def kernel(x, edge_index, W, b):
    n_edges = edge_index.shape[1]
    src = edge_index[0].astype(jnp.int32)
    dst = edge_index[1].astype(jnp.int32)
    n_extra = E_PAD - n_edges
    pad_idx = (jnp.arange(n_extra, dtype=jnp.int32) % (N_PAD - N_NODES)) + N_NODES
    src2d = jnp.concatenate([src, pad_idx]).reshape(NW * CPW, CHUNK)
    dst2d = jnp.concatenate([dst, pad_idx]).reshape(NW * CPW, CHUNK)
    x_pad = jnp.concatenate([x, jnp.zeros((N_PAD - N_NODES, DIM), x.dtype)])
    zrow = jnp.zeros((RPT,), jnp.float32)
    zblk = jnp.zeros((CHUNK, DIM), jnp.float32)

    degp = _deg(dst2d, zrow)
    g = _lin(x_pad, W, degp)
    accp = _agg(g, src2d, dst2d, zblk)
    out_pad = _out(accp, g, degp, b.reshape(1, DIM))
    return out_pad[:N_NODES]
